# baseline (speedup 1.0000x reference)
import jax
import jax.numpy as jnp
from jax.experimental import pallas as pl
from jax.experimental.pallas import tpu as pltpu

_NBUF = 8
_CH = 2


def _body(x_hbm, pos_vmem, o_hbm, ibuf, obuf, in_sems, out_sems):
    n_chunks = x_hbm.shape[0] // _CH

    def in_copy(c, slot):
        return pltpu.make_async_copy(
            x_hbm.at[pl.ds(c * _CH, _CH)], ibuf.at[slot], in_sems.at[slot]
        )

    def out_copy(c, slot):
        return pltpu.make_async_copy(
            obuf.at[slot], o_hbm.at[pl.ds(c * _CH, _CH)], out_sems.at[slot]
        )

    for k in range(_NBUF):
        in_copy(k, k).start()

    def step(c, carry):
        slot = jax.lax.rem(c, _NBUF)
        in_copy(c, slot).wait()

        @pl.when(c >= _NBUF)
        def _():
            out_copy(c - _NBUF, slot).wait()

        obuf[slot] = ibuf[slot] + pos_vmem[pl.ds(c * _CH, _CH), :][:, :, None]
        out_copy(c, slot).start()

        @pl.when(c + _NBUF < n_chunks)
        def _():
            in_copy(c + _NBUF, slot).start()

        return carry

    jax.lax.fori_loop(0, n_chunks, step, 0)

    for k in range(_NBUF):
        c = n_chunks - _NBUF + k
        out_copy(c, jax.lax.rem(c, _NBUF)).wait()


def kernel(x, pos_table):
    B, L, D = x.shape
    xt = jnp.transpose(x, (1, 2, 0))
    pos = jax.lax.slice(pos_table, (0, 0), (L, D))
    out_t = pl.pallas_call(
        _body,
        in_specs=[
            pl.BlockSpec(memory_space=pltpu.HBM),
            pl.BlockSpec(memory_space=pltpu.VMEM),
        ],
        out_specs=pl.BlockSpec(memory_space=pltpu.HBM),
        out_shape=jax.ShapeDtypeStruct((L, D, B), x.dtype),
        scratch_shapes=[
            pltpu.VMEM((_NBUF, _CH, D, B), jnp.float32),
            pltpu.VMEM((_NBUF, _CH, D, B), jnp.float32),
            pltpu.SemaphoreType.DMA((_NBUF,)),
            pltpu.SemaphoreType.DMA((_NBUF,)),
        ],
    )(xt, pos)
    return jnp.transpose(out_t, (2, 0, 1))
